# baseline (speedup 1.0000x reference)
import dataclasses
import functools

import jax
import jax.numpy as jnp
from jax import lax
from jax.experimental import pallas as pl
from jax.experimental.pallas import tpu as pltpu
from jax.experimental.pallas import tpu_sc as plsc

N = 10000
E = 320000
D = 128

NUM_CORES = 2
NUM_SUBCORES = 16
NUM_WORKERS = NUM_CORES * NUM_SUBCORES
NPAD = 10240
LANES = 16
BLK = 1024

CHUNK = 128
NCHUNKS = E // CHUNK
CHUNKS_LO = NCHUNKS // NUM_WORKERS
CHUNKS_HI = CHUNKS_LO + 1
NUM_HI = NCHUNKS - CHUNKS_LO * NUM_WORKERS
EPW_HI = CHUNKS_HI * CHUNK
EPW_LO = CHUNKS_LO * CHUNK


def _sc_compiler_params():
    cp = pltpu.CompilerParams()
    if "needs_layout_passes" in pltpu.CompilerParams.__dataclass_fields__:
        cp = dataclasses.replace(cp, needs_layout_passes=False)
    return cp


@jax.jit
def _sc_partial_counts(edge_index):
    mesh = plsc.VectorSubcoreMesh(core_axis_name="c", subcore_axis_name="s")

    @functools.partial(
        pl.kernel,
        out_type=jax.ShapeDtypeStruct((NUM_WORKERS, NPAD), jnp.float32),
        mesh=mesh,
        scratch_types=[
            pltpu.VMEM((2, EPW_HI), jnp.int32),
            pltpu.VMEM((NPAD,), jnp.float32),
            pltpu.SemaphoreType.DMA,
        ],
        compiler_params=_sc_compiler_params(),
    )
    def hist_kernel(edge_hbm, out_hbm, idx_vmem, hist_vmem, sem):
        wid = lax.axis_index("c") * NUM_SUBCORES + lax.axis_index("s")
        is_hi = wid < NUM_HI
        start = jnp.where(
            is_hi, wid * EPW_HI, NUM_HI * EPW_HI + (wid - NUM_HI) * EPW_LO
        )
        nchunks = jnp.where(is_hi, CHUNKS_HI, CHUNKS_LO)

        @pl.when(is_hi)
        def _():
            pltpu.make_async_copy(
                edge_hbm.at[:, pl.ds(start, EPW_HI)], idx_vmem, sem
            ).start()

        @pl.when(jnp.logical_not(is_hi))
        def _():
            pltpu.make_async_copy(
                edge_hbm.at[:, pl.ds(start, EPW_LO)],
                idx_vmem.at[:, pl.ds(0, EPW_LO)],
                sem,
            ).start()

        zeros = jnp.zeros((LANES,), jnp.float32)
        z_unroll = 8

        @pl.loop(0, NPAD // LANES, step=z_unroll)
        def _(i):
            for u in range(z_unroll):
                hist_vmem[pl.ds((i + u) * LANES, LANES)] = zeros

        @pl.when(is_hi)
        def _():
            pltpu.make_async_copy(
                edge_hbm.at[:, pl.ds(start, EPW_HI)], idx_vmem, sem
            ).wait()

        @pl.when(jnp.logical_not(is_hi))
        def _():
            pltpu.make_async_copy(
                edge_hbm.at[:, pl.ds(start, EPW_LO)],
                idx_vmem.at[:, pl.ds(0, EPW_LO)],
                sem,
            ).wait()

        ones = jnp.full((LANES,), 1.0, jnp.float32)
        vecs_per_chunk = CHUNK // LANES

        @pl.loop(0, nchunks)
        def _(c):
            for u in range(vecs_per_chunk):
                iv = idx_vmem[0, pl.ds(c * CHUNK + u * LANES, LANES)]
                plsc.addupdate_scatter(hist_vmem, [iv], ones)

        pltpu.sync_copy(hist_vmem, out_hbm.at[wid])

    return hist_kernel(edge_index)


def _tc_fused_body(partial_ref, atom_ref, h_ref, hout_ref, viol_ref):
    counts = jnp.sum(partial_ref[...], axis=0, keepdims=True)
    at = atom_ref[...]
    maxv = jnp.where(
        at == 1, 3.0,
        jnp.where(
            at == 2, 2.0,
            jnp.where(at == 3, 6.0, jnp.where((at >= 4) & (at <= 7), 1.0, 4.0)),
        ),
    ).astype(jnp.float32)
    mask = jnp.maximum(counts - maxv, 0.0)

    viol_ref[...] = (jnp.sum(mask * mask) / jnp.float32(N)).reshape(1, 1, 1)

    scale = 1.0 - 0.1 * mask
    eye_row = lax.broadcasted_iota(jnp.int32, (D, D), 0)
    eye_col = lax.broadcasted_iota(jnp.int32, (D, D), 1)
    eye = (eye_row == eye_col).astype(jnp.float32)
    blocks = []
    for g in range(BLK // D):
        sg = scale[:, g * D:(g + 1) * D]
        blocks.append(jnp.broadcast_to(sg, (D, D)) * eye)
    dmat = jnp.concatenate(blocks, axis=0)
    smat = jnp.dot(dmat, jnp.ones((D, D), jnp.float32),
                   preferred_element_type=jnp.float32)
    hout_ref[...] = h_ref[...] * smat


@jax.jit
def _tc_finish(partial, h, atom_row):
    grid = NPAD // BLK
    return pl.pallas_call(
        _tc_fused_body,
        grid=(grid,),
        in_specs=[
            pl.BlockSpec((NUM_WORKERS, BLK), lambda i: (0, i)),
            pl.BlockSpec((1, BLK), lambda i: (0, i)),
            pl.BlockSpec((BLK, D), lambda i: (i, 0)),
        ],
        out_specs=[
            pl.BlockSpec((BLK, D), lambda i: (i, 0)),
            pl.BlockSpec((1, 1, 1), lambda i: (i, 0, 0)),
        ],
        out_shape=[
            jax.ShapeDtypeStruct((N, D), jnp.float32),
            jax.ShapeDtypeStruct((NPAD // BLK, 1, 1), jnp.float32),
        ],
        compiler_params=pltpu.CompilerParams(
            dimension_semantics=("parallel",)
        ),
    )(partial, atom_row, h)


def kernel(h, edge_index, predicted_valences, atom_types):
    del predicted_valences
    partial = _sc_partial_counts(edge_index)
    atom_row = jnp.pad(atom_types, (0, NPAD - N)).reshape(1, NPAD)
    h_out, viol_parts = _tc_finish(partial, h, atom_row)
    return (h_out, jnp.sum(viol_parts).reshape(()))


# speedup vs baseline: 1.2854x; 1.2854x over previous
import dataclasses
import functools

import jax
import jax.numpy as jnp
from jax import lax
from jax.experimental import pallas as pl
from jax.experimental.pallas import tpu as pltpu
from jax.experimental.pallas import tpu_sc as plsc

N = 10000
E = 320000
D = 128

NUM_CORES = 2
NUM_SUBCORES = 16
NUM_WORKERS = NUM_CORES * NUM_SUBCORES
NPAD = 10240
LANES = 16
BLK = 5120

CHUNK = 128
NCHUNKS = E // CHUNK
CHUNKS_LO = NCHUNKS // NUM_WORKERS
CHUNKS_HI = CHUNKS_LO + 1
NUM_HI = NCHUNKS - CHUNKS_LO * NUM_WORKERS
EPW_HI = CHUNKS_HI * CHUNK
EPW_LO = CHUNKS_LO * CHUNK


def _sc_compiler_params():
    cp = pltpu.CompilerParams()
    if "needs_layout_passes" in pltpu.CompilerParams.__dataclass_fields__:
        cp = dataclasses.replace(cp, needs_layout_passes=False)
    return cp


@jax.jit
def _sc_partial_counts(edge_index):
    mesh = plsc.VectorSubcoreMesh(core_axis_name="c", subcore_axis_name="s")

    @functools.partial(
        pl.kernel,
        out_type=jax.ShapeDtypeStruct((NUM_WORKERS, NPAD), jnp.float32),
        mesh=mesh,
        scratch_types=[
            pltpu.VMEM((2, EPW_HI), jnp.int32),
            pltpu.VMEM((NPAD,), jnp.float32),
            pltpu.SemaphoreType.DMA,
        ],
        compiler_params=_sc_compiler_params(),
    )
    def hist_kernel(edge_hbm, out_hbm, idx_vmem, hist_vmem, sem):
        wid = lax.axis_index("c") * NUM_SUBCORES + lax.axis_index("s")
        is_hi = wid < NUM_HI
        start = jnp.where(
            is_hi, wid * EPW_HI, NUM_HI * EPW_HI + (wid - NUM_HI) * EPW_LO
        )
        nchunks = jnp.where(is_hi, CHUNKS_HI, CHUNKS_LO)

        @pl.when(is_hi)
        def _():
            pltpu.make_async_copy(
                edge_hbm.at[:, pl.ds(start, EPW_HI)], idx_vmem, sem
            ).start()

        @pl.when(jnp.logical_not(is_hi))
        def _():
            pltpu.make_async_copy(
                edge_hbm.at[:, pl.ds(start, EPW_LO)],
                idx_vmem.at[:, pl.ds(0, EPW_LO)],
                sem,
            ).start()

        zeros = jnp.zeros((LANES,), jnp.float32)

        @plsc.parallel_loop(0, NPAD // LANES, unroll=8)
        def _(i):
            hist_vmem[pl.ds(i * LANES, LANES)] = zeros

        @pl.when(is_hi)
        def _():
            pltpu.make_async_copy(
                edge_hbm.at[:, pl.ds(start, EPW_HI)], idx_vmem, sem
            ).wait()

        @pl.when(jnp.logical_not(is_hi))
        def _():
            pltpu.make_async_copy(
                edge_hbm.at[:, pl.ds(start, EPW_LO)],
                idx_vmem.at[:, pl.ds(0, EPW_LO)],
                sem,
            ).wait()

        ones = jnp.full((LANES,), 1.0, jnp.float32)
        vecs_per_chunk = CHUNK // LANES

        @plsc.parallel_loop(0, nchunks, unroll=4)
        def _(c):
            for u in range(vecs_per_chunk):
                iv = idx_vmem[0, pl.ds(c * CHUNK + u * LANES, LANES)]
                plsc.addupdate_scatter(hist_vmem, [iv], ones)

        pltpu.sync_copy(hist_vmem, out_hbm.at[wid])

    return hist_kernel(edge_index)


def _tc_fused_body(partial_ref, atom_ref, h_ref, hout_ref, viol_ref):
    counts = jnp.sum(partial_ref[...], axis=0, keepdims=True)
    at = atom_ref[...]
    maxv = jnp.where(
        at == 1, 3.0,
        jnp.where(
            at == 2, 2.0,
            jnp.where(at == 3, 6.0, jnp.where((at >= 4) & (at <= 7), 1.0, 4.0)),
        ),
    ).astype(jnp.float32)
    mask = jnp.maximum(counts - maxv, 0.0)

    i = pl.program_id(0)

    @pl.when(i == 0)
    def _():
        viol_ref[...] = jnp.zeros_like(viol_ref)

    viol_ref[...] += (jnp.sum(mask * mask) / jnp.float32(N)).reshape(1, 1)

    scale = 1.0 - 0.1 * mask
    eye_row = lax.broadcasted_iota(jnp.int32, (D, D), 0)
    eye_col = lax.broadcasted_iota(jnp.int32, (D, D), 1)
    eye = (eye_row == eye_col).astype(jnp.float32)
    blocks = []
    for g in range(BLK // D):
        sg = scale[:, g * D:(g + 1) * D]
        blocks.append(jnp.broadcast_to(sg, (D, D)) * eye)
    dmat = jnp.concatenate(blocks, axis=0)
    smat = jnp.dot(dmat, jnp.ones((D, D), jnp.float32),
                   preferred_element_type=jnp.float32)
    hout_ref[...] = h_ref[...] * smat


@jax.jit
def _tc_finish(partial, h, atom_row):
    grid = NPAD // BLK
    return pl.pallas_call(
        _tc_fused_body,
        grid=(grid,),
        in_specs=[
            pl.BlockSpec((NUM_WORKERS, BLK), lambda i: (0, i)),
            pl.BlockSpec((1, BLK), lambda i: (0, i)),
            pl.BlockSpec((BLK, D), lambda i: (i, 0)),
        ],
        out_specs=[
            pl.BlockSpec((BLK, D), lambda i: (i, 0)),
            pl.BlockSpec((1, 1), lambda i: (0, 0)),
        ],
        out_shape=[
            jax.ShapeDtypeStruct((N, D), jnp.float32),
            jax.ShapeDtypeStruct((1, 1), jnp.float32),
        ],
    )(partial, atom_row, h)


def kernel(h, edge_index, predicted_valences, atom_types):
    del predicted_valences
    partial = _sc_partial_counts(edge_index)
    atom_row = jnp.pad(atom_types, (0, NPAD - N)).reshape(1, NPAD)
    h_out, viol = _tc_finish(partial, h, atom_row)
    return (h_out, viol.reshape(()))
